# baseline (speedup 1.0000x reference)
import jax
import jax.numpy as jnp
from jax import lax
from jax.experimental import pallas as pl
from jax.experimental.pallas import tpu as pltpu
from jax.experimental.pallas import tpu_sc as plsc

N = 10000
E = 320000
D = 128
ALPHA = 0.2

NC = 2
NS = 16
NW = NC * NS
EPW = E // NW
K = 80
NCHUNK = EPW // K
RPT = 640
RPT_LAST = N - (NS - 1) * RPT


def _stage1_body(x_ref, w_ref, a_ref, h_ref, f12_ref):
    h = jnp.dot(x_ref[...], w_ref[...], preferred_element_type=jnp.float32)
    h_ref[...] = h
    a1 = a_ref[0:1, :D]
    a2 = a_ref[0:1, D:]
    f1 = jnp.sum(h * a1, axis=1, keepdims=True)
    f2 = jnp.sum(h * a2, axis=1, keepdims=True)
    u1 = lax.bitcast_convert_type(
        f1.astype(jnp.bfloat16), jnp.uint16).astype(jnp.uint32)
    u2 = lax.bitcast_convert_type(
        f2.astype(jnp.bfloat16), jnp.uint16).astype(jnp.uint32)
    f12_ref[...] = lax.bitcast_convert_type((u2 << 16) | u1, jnp.int32)


def _stage1(x, W, a):
    blk = 2000
    return pl.pallas_call(
        _stage1_body,
        grid=(N // blk,),
        in_specs=[
            pl.BlockSpec((blk, D), lambda i: (i, 0)),
            pl.BlockSpec((D, D), lambda i: (0, 0)),
            pl.BlockSpec((1, 2 * D), lambda i: (0, 0)),
        ],
        out_specs=[
            pl.BlockSpec((blk, D), lambda i: (i, 0)),
            pl.BlockSpec((blk, 1), lambda i: (i, 0)),
        ],
        out_shape=[
            jax.ShapeDtypeStruct((N, D), jnp.float32),
            jax.ShapeDtypeStruct((N, 1), jnp.int32),
        ],
    )(x, W, a)


RSROWS = 80


def _stage2(h, eidx, f12, zrs):
    mesh = plsc.VectorSubcoreMesh(core_axis_name="c", subcore_axis_name="s")

    def body(h_r, eidx_r, f12_r, zrs_r, parts_r, rsp_r,
             f12_v, rows0_v, rows1_v, rs_v, wc0_v, wc1_v, idx0_v, idx1_v,
             sd0_v, sd1_v, acc_sh, rs_sh,
             sem_i0, sem_i1, sem_g0, sem_g1, sem_s0, sem_s1):
        cid = lax.axis_index("c")
        sid = lax.axis_index("s")
        wid = cid * NS + sid

        pltpu.sync_copy(f12_r, f12_v)

        @pl.when(sid < NS - 1)
        def _():
            base = pl.multiple_of(sid * RPT, 8)
            pltpu.sync_copy(zrs_r, acc_sh.at[pl.ds(base, RPT)])

        @pl.when(sid == NS - 1)
        def _():
            pltpu.sync_copy(zrs_r.at[pl.ds(0, RPT_LAST)],
                            acc_sh.at[pl.ds((NS - 1) * RPT, RPT_LAST)])

        @pl.when(sid == 0)
        def _():
            pltpu.sync_copy(zrs_r.at[pl.ds(0, RSROWS)], rs_sh)

        zero16 = jnp.zeros((16,), jnp.float32)

        def zbody(i, c):
            for j in range(D // 16):
                rs_v[i, pl.ds(j * 16, 16)] = zero16
            return c
        lax.fori_loop(0, RSROWS, zbody, 0)

        ebase0 = wid * EPW * 2

        def fire_fill(c, idx_v, sem_i):
            off = pl.multiple_of(ebase0 + c * 2 * K, 8)
            return pltpu.async_copy(
                eidx_r.at[pl.ds(off, 2 * K)], idx_v, sem_i)

        fire_fill(0, idx0_v, sem_i0)
        fire_fill(1, idx1_v, sem_i1)

        plsc.subcore_barrier()

        bufs = [
            (idx0_v, sd0_v, rows0_v, wc0_v, sem_i0, sem_g0, sem_s0),
            (idx1_v, sd1_v, rows1_v, wc1_v, sem_i1, sem_g1, sem_s1),
        ]

        def prepare(c, b, steady):
            idx_v, sd_v, rows_v, wc_v, sem_i, sem_g, sem_s = bufs[b]

            pltpu.make_async_copy(
                eidx_r.at[pl.ds(0, 2 * K)], idx_v, sem_i).wait()

            if steady:
                pltpu.make_async_copy(
                    h_r.at[pl.ds(0, K)], rows_v, sem_s).wait()

            for g in range(K // 16):
                sd_v[0, pl.ds(g * 16, 16)] = idx_v[pl.ds(g * 16, 16)]
                sd_v[1, pl.ds(g * 16, 16)] = idx_v[pl.ds(K + g * 16, 16)]

            @pl.when(c < NCHUNK - 2)
            def _():
                fire_fill(c + 2, idx_v, sem_i)

            pltpu.async_copy(h_r.at[sd_v.at[1]], rows_v, sem_g)

            for g in range(K // 16):
                svec = sd_v[0, pl.ds(g * 16, 16)]
                dvec = sd_v[1, pl.ds(g * 16, 16)]
                gs = plsc.load_gather(f12_v, [svec])
                gd = plsc.load_gather(f12_v, [dvec])
                f1 = plsc.bitcast(gs << 16, jnp.float32)
                f2 = plsc.bitcast(gd & jnp.int32(-65536), jnp.float32)
                z = f1 + f2
                w = jnp.exp(-jnp.where(z > 0, z, ALPHA * z))
                wc_v[pl.ds(g * 16, 16)] = w
                plsc.addupdate_scatter(rs_v, [svec >> 7, svec & 127], w)

        def finish(c, b):
            del c
            idx_v, sd_v, rows_v, wc_v, sem_i, sem_g, sem_s = bufs[b]

            pltpu.make_async_copy(h_r.at[sd_v.at[1]], rows_v, sem_g).wait()

            for g in range(K // 16):
                wv = wc_v[pl.ds(g * 16, 16)]
                for l in range(16):
                    k = g * 16 + l
                    w = wv[l]
                    for j in range(D // 16):
                        rows_v[k, pl.ds(j * 16, 16)] = (
                            rows_v[k, pl.ds(j * 16, 16)] * w)

            pltpu.async_copy(rows_v, acc_sh.at[sd_v.at[0]], sem_s, add=True)

        prepare(0, 0, steady=False)
        prepare(1, 1, steady=False)

        def pair(p, c):
            c0 = p * 2
            finish(c0, 0)
            prepare(c0 + 2, 0, steady=True)
            finish(c0 + 1, 1)
            prepare(c0 + 3, 1, steady=True)
            return c
        lax.fori_loop(0, NCHUNK // 2 - 1, pair, 0)

        finish(NCHUNK - 3, 0)
        prepare(NCHUNK - 1, 0, steady=True)
        finish(NCHUNK - 2, 1)
        finish(NCHUNK - 1, 0)

        pltpu.make_async_copy(h_r.at[pl.ds(0, K)], rows0_v, sem_s0).wait()
        pltpu.make_async_copy(h_r.at[pl.ds(0, K)], rows1_v, sem_s1).wait()

        for g in range(RSROWS // 16):
            rowids = lax.iota(jnp.int32, 16) + g * 16
            pltpu.sync_copy(rs_v.at[pl.ds(g * 16, 16)],
                            rs_sh.at[rowids], add=True)

        plsc.subcore_barrier()

        @pl.when(sid == 0)
        def _():
            pltpu.sync_copy(rs_sh, rsp_r.at[cid])

        @pl.when(sid < NS - 1)
        def _():
            base = pl.multiple_of(sid * RPT, 8)
            pltpu.sync_copy(acc_sh.at[pl.ds(base, RPT)],
                            parts_r.at[cid, pl.ds(base, RPT)])

        @pl.when(sid == NS - 1)
        def _():
            pltpu.sync_copy(acc_sh.at[pl.ds((NS - 1) * RPT, RPT_LAST)],
                            parts_r.at[cid, pl.ds((NS - 1) * RPT, RPT_LAST)])

    f = pl.kernel(
        body,
        out_type=[
            jax.ShapeDtypeStruct((NC, N, D), jnp.float32),
            jax.ShapeDtypeStruct((NC, RSROWS, D), jnp.float32),
        ],
        mesh=mesh,
        scratch_types=[
            pltpu.VMEM((N,), jnp.int32),
            pltpu.VMEM((K, D), jnp.float32),
            pltpu.VMEM((K, D), jnp.float32),
            pltpu.VMEM((RSROWS, D), jnp.float32),
            pltpu.VMEM((K,), jnp.float32),
            pltpu.VMEM((K,), jnp.float32),
            pltpu.VMEM((2 * K,), jnp.int32),
            pltpu.VMEM((2 * K,), jnp.int32),
            pltpu.VMEM((2, K), jnp.int32),
            pltpu.VMEM((2, K), jnp.int32),
            pltpu.VMEM_SHARED((N, D), jnp.float32),
            pltpu.VMEM_SHARED((RSROWS, D), jnp.float32),
            pltpu.SemaphoreType.DMA,
            pltpu.SemaphoreType.DMA,
            pltpu.SemaphoreType.DMA,
            pltpu.SemaphoreType.DMA,
            pltpu.SemaphoreType.DMA,
            pltpu.SemaphoreType.DMA,
        ],
        compiler_params=pltpu.CompilerParams(needs_layout_passes=False),
    )
    return f(h, eidx, f12, zrs)


def _stage3_body(p0_ref, p1_ref, r0_ref, r1_ref, out_ref):
    hp = p0_ref[...] + p1_ref[...]
    rs = r0_ref[...] + r1_ref[...] + 1e-9
    r = hp / rs
    out_ref[...] = jnp.where(r > 0, r, jnp.exp(jnp.minimum(r, 0.0)) - 1.0)


def _stage3(p0, p1, r0, r1):
    blk = 2000
    return pl.pallas_call(
        _stage3_body,
        grid=(N // blk,),
        in_specs=[
            pl.BlockSpec((blk, D), lambda i: (i, 0)),
            pl.BlockSpec((blk, D), lambda i: (i, 0)),
            pl.BlockSpec((blk, 1), lambda i: (i, 0)),
            pl.BlockSpec((blk, 1), lambda i: (i, 0)),
        ],
        out_specs=pl.BlockSpec((blk, D), lambda i: (i, 0)),
        out_shape=jax.ShapeDtypeStruct((N, D), jnp.float32),
    )(p0, p1, r0, r1)


def kernel(input, edge_index, W, a):
    eidx = jnp.swapaxes(
        edge_index.reshape(2, NW * NCHUNK, K), 0, 1).reshape(-1)
    h, f12 = _stage1(input, W, a)
    zrs = jnp.zeros((RPT, D), jnp.float32)
    parts, rsp = _stage2(h, eidx, f12.reshape(N), zrs)
    rs = rsp.reshape(NC, RSROWS * D)[:, :N].reshape(NC, N, 1)
    return _stage3(parts[0], parts[1], rs[0], rs[1])


# speedup vs baseline: 1.3135x; 1.3135x over previous
import jax
import jax.numpy as jnp
from jax import lax
from jax.experimental import pallas as pl
from jax.experimental.pallas import tpu as pltpu
from jax.experimental.pallas import tpu_sc as plsc

N = 10000
E = 320000
D = 128
ALPHA = 0.2

NC = 2
NS = 16
NW = NC * NS
EPW = E // NW
K = 80
NCHUNK = EPW // K
RPT = 640
RPT_LAST = N - (NS - 1) * RPT


def _stage1_body(x_ref, w_ref, a_ref, h_ref, f12_ref):
    h = jnp.dot(x_ref[...], w_ref[...], preferred_element_type=jnp.float32)
    h_ref[...] = h
    a1 = a_ref[0:1, :D]
    a2 = a_ref[0:1, D:]
    f1 = jnp.sum(h * a1, axis=1, keepdims=True)
    f2 = jnp.sum(h * a2, axis=1, keepdims=True)
    u1 = lax.bitcast_convert_type(
        f1.astype(jnp.bfloat16), jnp.uint16).astype(jnp.uint32)
    u2 = lax.bitcast_convert_type(
        f2.astype(jnp.bfloat16), jnp.uint16).astype(jnp.uint32)
    f12_ref[...] = lax.bitcast_convert_type((u2 << 16) | u1, jnp.int32)


def _stage1(x, W, a):
    blk = 2000
    return pl.pallas_call(
        _stage1_body,
        grid=(N // blk,),
        in_specs=[
            pl.BlockSpec((blk, D), lambda i: (i, 0)),
            pl.BlockSpec((D, D), lambda i: (0, 0)),
            pl.BlockSpec((1, 2 * D), lambda i: (0, 0)),
        ],
        out_specs=[
            pl.BlockSpec((blk, D), lambda i: (i, 0)),
            pl.BlockSpec((blk, 1), lambda i: (i, 0)),
        ],
        out_shape=[
            jax.ShapeDtypeStruct((N, D), jnp.float32),
            jax.ShapeDtypeStruct((N, 1), jnp.int32),
        ],
    )(x, W, a)


RSROWS = 80


def _stage2(h, eidx, f12, zrs):
    mesh = plsc.VectorSubcoreMesh(core_axis_name="c", subcore_axis_name="s")

    def body(h_r, eidx_r, f12_r, zrs_r, parts_r, rsp_r,
             f12_v, rows0_v, rows1_v, rs_v, wc0_v, wc1_v, idx0_v, idx1_v,
             sd0_v, sd1_v, acc_sh, rs_sh,
             sem_i0, sem_i1, sem_g0, sem_g1, sem_s0, sem_s1):
        cid = lax.axis_index("c")
        sid = lax.axis_index("s")
        wid = cid * NS + sid

        pltpu.sync_copy(f12_r, f12_v)

        @pl.when(sid < NS - 1)
        def _():
            base = pl.multiple_of(sid * RPT, 8)
            pltpu.sync_copy(zrs_r, acc_sh.at[pl.ds(base, RPT)])

        @pl.when(sid == NS - 1)
        def _():
            pltpu.sync_copy(zrs_r.at[pl.ds(0, RPT_LAST)],
                            acc_sh.at[pl.ds((NS - 1) * RPT, RPT_LAST)])

        @pl.when(sid == 0)
        def _():
            pltpu.sync_copy(zrs_r.at[pl.ds(0, RSROWS)], rs_sh)

        zero16 = jnp.zeros((16,), jnp.float32)

        def zbody(i, c):
            for j in range(D // 16):
                rs_v[i, pl.ds(j * 16, 16)] = zero16
            return c
        lax.fori_loop(0, RSROWS, zbody, 0)

        ebase0 = wid * EPW * 2

        def fire_fill(c, idx_v, sem_i):
            off = pl.multiple_of(ebase0 + c * 2 * K, 8)
            return pltpu.async_copy(
                eidx_r.at[pl.ds(off, 2 * K)], idx_v, sem_i)

        fire_fill(0, idx0_v, sem_i0)
        fire_fill(1, idx1_v, sem_i1)

        plsc.subcore_barrier()

        bufs = [
            (idx0_v, sd0_v, rows0_v, wc0_v, sem_i0, sem_g0, sem_s0),
            (idx1_v, sd1_v, rows1_v, wc1_v, sem_i1, sem_g1, sem_s1),
        ]

        def prepare(c, b, steady):
            idx_v, sd_v, rows_v, wc_v, sem_i, sem_g, sem_s = bufs[b]

            pltpu.make_async_copy(
                eidx_r.at[pl.ds(0, 2 * K)], idx_v, sem_i).wait()

            if steady:
                pltpu.make_async_copy(
                    h_r.at[pl.ds(0, K)], rows_v, sem_s).wait()

            for g in range(K // 16):
                sd_v[0, pl.ds(g * 16, 16)] = idx_v[pl.ds(g * 16, 16)]
                sd_v[1, pl.ds(g * 16, 16)] = idx_v[pl.ds(K + g * 16, 16)]

            @pl.when(c < NCHUNK - 2)
            def _():
                fire_fill(c + 2, idx_v, sem_i)

            pltpu.async_copy(h_r.at[sd_v.at[1]], rows_v, sem_g)

            for g in range(K // 16):
                svec = sd_v[0, pl.ds(g * 16, 16)]
                dvec = sd_v[1, pl.ds(g * 16, 16)]
                gs = plsc.load_gather(f12_v, [svec])
                gd = plsc.load_gather(f12_v, [dvec])
                f1 = plsc.bitcast(gs << 16, jnp.float32)
                f2 = plsc.bitcast(gd & jnp.int32(-65536), jnp.float32)
                z = f1 + f2
                w = jnp.exp(-jnp.where(z > 0, z, ALPHA * z))
                wc_v[pl.ds(g * 16, 16)] = w
                plsc.addupdate_scatter(rs_v, [svec >> 7, svec & 127], w)

        def finish(c, b):
            del c
            idx_v, sd_v, rows_v, wc_v, sem_i, sem_g, sem_s = bufs[b]

            pltpu.make_async_copy(h_r.at[sd_v.at[1]], rows_v, sem_g).wait()

            @plsc.parallel_loop(0, K // 16)
            def scale(g):
                wv = wc_v[pl.ds(g * 16, 16)]
                for l in range(16):
                    k = g * 16 + l
                    w = wv[l]
                    for j in range(D // 16):
                        rows_v[k, pl.ds(j * 16, 16)] = (
                            rows_v[k, pl.ds(j * 16, 16)] * w)

            pltpu.async_copy(rows_v, acc_sh.at[sd_v.at[0]], sem_s, add=True)

        prepare(0, 0, steady=False)
        prepare(1, 1, steady=False)

        def pair(p, c):
            c0 = p * 2
            finish(c0, 0)
            prepare(c0 + 2, 0, steady=True)
            finish(c0 + 1, 1)
            prepare(c0 + 3, 1, steady=True)
            return c
        lax.fori_loop(0, NCHUNK // 2 - 1, pair, 0)

        finish(NCHUNK - 3, 0)
        prepare(NCHUNK - 1, 0, steady=True)
        finish(NCHUNK - 2, 1)
        finish(NCHUNK - 1, 0)

        pltpu.make_async_copy(h_r.at[pl.ds(0, K)], rows0_v, sem_s0).wait()
        pltpu.make_async_copy(h_r.at[pl.ds(0, K)], rows1_v, sem_s1).wait()

        for g in range(RSROWS // 16):
            rowids = lax.iota(jnp.int32, 16) + g * 16
            pltpu.sync_copy(rs_v.at[pl.ds(g * 16, 16)],
                            rs_sh.at[rowids], add=True)

        plsc.subcore_barrier()

        @pl.when(sid == 0)
        def _():
            pltpu.sync_copy(rs_sh, rsp_r.at[cid])

        @pl.when(sid < NS - 1)
        def _():
            base = pl.multiple_of(sid * RPT, 8)
            pltpu.sync_copy(acc_sh.at[pl.ds(base, RPT)],
                            parts_r.at[cid, pl.ds(base, RPT)])

        @pl.when(sid == NS - 1)
        def _():
            pltpu.sync_copy(acc_sh.at[pl.ds((NS - 1) * RPT, RPT_LAST)],
                            parts_r.at[cid, pl.ds((NS - 1) * RPT, RPT_LAST)])

    f = pl.kernel(
        body,
        out_type=[
            jax.ShapeDtypeStruct((NC, N, D), jnp.float32),
            jax.ShapeDtypeStruct((NC, RSROWS, D), jnp.float32),
        ],
        mesh=mesh,
        scratch_types=[
            pltpu.VMEM((N,), jnp.int32),
            pltpu.VMEM((K, D), jnp.float32),
            pltpu.VMEM((K, D), jnp.float32),
            pltpu.VMEM((RSROWS, D), jnp.float32),
            pltpu.VMEM((K,), jnp.float32),
            pltpu.VMEM((K,), jnp.float32),
            pltpu.VMEM((2 * K,), jnp.int32),
            pltpu.VMEM((2 * K,), jnp.int32),
            pltpu.VMEM((2, K), jnp.int32),
            pltpu.VMEM((2, K), jnp.int32),
            pltpu.VMEM_SHARED((N, D), jnp.float32),
            pltpu.VMEM_SHARED((RSROWS, D), jnp.float32),
            pltpu.SemaphoreType.DMA,
            pltpu.SemaphoreType.DMA,
            pltpu.SemaphoreType.DMA,
            pltpu.SemaphoreType.DMA,
            pltpu.SemaphoreType.DMA,
            pltpu.SemaphoreType.DMA,
        ],
        compiler_params=pltpu.CompilerParams(needs_layout_passes=False),
    )
    return f(h, eidx, f12, zrs)


def _stage3_body(p0_ref, p1_ref, r0_ref, r1_ref, out_ref):
    hp = p0_ref[...] + p1_ref[...]
    rs = r0_ref[...] + r1_ref[...] + 1e-9
    r = hp / rs
    out_ref[...] = jnp.where(r > 0, r, jnp.exp(jnp.minimum(r, 0.0)) - 1.0)


def _stage3(p0, p1, r0, r1):
    blk = 2000
    return pl.pallas_call(
        _stage3_body,
        grid=(N // blk,),
        in_specs=[
            pl.BlockSpec((blk, D), lambda i: (i, 0)),
            pl.BlockSpec((blk, D), lambda i: (i, 0)),
            pl.BlockSpec((blk, 1), lambda i: (i, 0)),
            pl.BlockSpec((blk, 1), lambda i: (i, 0)),
        ],
        out_specs=pl.BlockSpec((blk, D), lambda i: (i, 0)),
        out_shape=jax.ShapeDtypeStruct((N, D), jnp.float32),
    )(p0, p1, r0, r1)


def kernel(input, edge_index, W, a):
    eidx = jnp.swapaxes(
        edge_index.reshape(2, NW * NCHUNK, K), 0, 1).reshape(-1)
    h, f12 = _stage1(input, W, a)
    zrs = jnp.zeros((RPT, D), jnp.float32)
    parts, rsp = _stage2(h, eidx, f12.reshape(N), zrs)
    rs = rsp.reshape(NC, RSROWS * D)[:, :N].reshape(NC, N, 1)
    return _stage3(parts[0], parts[1], rs[0], rs[1])


# speedup vs baseline: 1.3150x; 1.0011x over previous
import jax
import jax.numpy as jnp
from jax import lax
from jax.experimental import pallas as pl
from jax.experimental.pallas import tpu as pltpu
from jax.experimental.pallas import tpu_sc as plsc

N = 10000
E = 320000
D = 128
ALPHA = 0.2

NC = 2
NS = 16
NW = NC * NS
EPW = E // NW
K = 80
NCHUNK = EPW // K
RPT = 640
RPT_LAST = N - (NS - 1) * RPT


def _stage1_body(x_ref, w_ref, a_ref, h_ref, f12_ref):
    h = jnp.dot(x_ref[...], w_ref[...], preferred_element_type=jnp.float32)
    h_ref[...] = h
    a1 = a_ref[0:1, :D]
    a2 = a_ref[0:1, D:]
    f1 = jnp.sum(h * a1, axis=1, keepdims=True)
    f2 = jnp.sum(h * a2, axis=1, keepdims=True)
    u1 = lax.bitcast_convert_type(
        f1.astype(jnp.bfloat16), jnp.uint16).astype(jnp.uint32)
    u2 = lax.bitcast_convert_type(
        f2.astype(jnp.bfloat16), jnp.uint16).astype(jnp.uint32)
    f12_ref[...] = lax.bitcast_convert_type((u2 << 16) | u1, jnp.int32)


def _stage1(x, W, a):
    blk = 2000
    return pl.pallas_call(
        _stage1_body,
        grid=(N // blk,),
        in_specs=[
            pl.BlockSpec((blk, D), lambda i: (i, 0)),
            pl.BlockSpec((D, D), lambda i: (0, 0)),
            pl.BlockSpec((1, 2 * D), lambda i: (0, 0)),
        ],
        out_specs=[
            pl.BlockSpec((blk, D), lambda i: (i, 0)),
            pl.BlockSpec((blk, 1), lambda i: (i, 0)),
        ],
        out_shape=[
            jax.ShapeDtypeStruct((N, D), jnp.float32),
            jax.ShapeDtypeStruct((N, 1), jnp.int32),
        ],
    )(x, W, a)


RSROWS = 80


def _stage2(h, eidx, f12, zrs):
    mesh = plsc.VectorSubcoreMesh(core_axis_name="c", subcore_axis_name="s")

    def body(h_r, eidx_r, f12_r, zrs_r, parts_r, rsp_r,
             f12_v, rows0_v, rows1_v, rs_v, wc0_v, wc1_v, idx0_v, idx1_v,
             sd0_v, sd1_v, acc_sh, rs_sh,
             sem_i0, sem_i1, sem_g0, sem_g1, sem_s0, sem_s1):
        cid = lax.axis_index("c")
        sid = lax.axis_index("s")
        wid = cid * NS + sid

        pltpu.sync_copy(f12_r, f12_v)

        @pl.when(sid < NS - 1)
        def _():
            base = pl.multiple_of(sid * RPT, 8)
            pltpu.sync_copy(zrs_r, acc_sh.at[pl.ds(base, RPT)])

        @pl.when(sid == NS - 1)
        def _():
            pltpu.sync_copy(zrs_r.at[pl.ds(0, RPT_LAST)],
                            acc_sh.at[pl.ds((NS - 1) * RPT, RPT_LAST)])

        @pl.when(sid == 0)
        def _():
            pltpu.sync_copy(zrs_r.at[pl.ds(0, RSROWS)], rs_sh)

        zero16 = jnp.zeros((16,), jnp.float32)

        def zbody(i, c):
            for j in range(D // 16):
                rs_v[i, pl.ds(j * 16, 16)] = zero16
            return c
        lax.fori_loop(0, RSROWS, zbody, 0)

        ebase0 = wid * EPW * 2

        def fire_fill(c, idx_v, sem_i):
            off = pl.multiple_of(ebase0 + c * 2 * K, 8)
            return pltpu.async_copy(
                eidx_r.at[pl.ds(off, 2 * K)], idx_v, sem_i)

        fire_fill(0, idx0_v, sem_i0)
        fire_fill(1, idx1_v, sem_i1)

        plsc.subcore_barrier()

        bufs = [
            (idx0_v, sd0_v, rows0_v, wc0_v, sem_i0, sem_g0, sem_s0),
            (idx1_v, sd1_v, rows1_v, wc1_v, sem_i1, sem_g1, sem_s1),
        ]

        def prepare(c, b, steady):
            idx_v, sd_v, rows_v, wc_v, sem_i, sem_g, sem_s = bufs[b]

            pltpu.make_async_copy(
                eidx_r.at[pl.ds(0, 2 * K)], idx_v, sem_i).wait()

            if steady:
                pltpu.make_async_copy(
                    h_r.at[pl.ds(0, K)], rows_v, sem_s).wait()

            for g in range(K // 16):
                sd_v[0, pl.ds(g * 16, 16)] = idx_v[pl.ds(g * 16, 16)]
                sd_v[1, pl.ds(g * 16, 16)] = idx_v[pl.ds(K + g * 16, 16)]

            @pl.when(c < NCHUNK - 2)
            def _():
                fire_fill(c + 2, idx_v, sem_i)

            pltpu.async_copy(h_r.at[sd_v.at[1]], rows_v, sem_g)

            for g in range(K // 16):
                svec = sd_v[0, pl.ds(g * 16, 16)]
                dvec = sd_v[1, pl.ds(g * 16, 16)]
                gs = plsc.load_gather(f12_v, [svec])
                gd = plsc.load_gather(f12_v, [dvec])
                f1 = plsc.bitcast(gs << 16, jnp.float32)
                f2 = plsc.bitcast(gd & jnp.int32(-65536), jnp.float32)
                z = f1 + f2
                w = jnp.exp(-jnp.where(z > 0, z, ALPHA * z))
                wc_v[pl.ds(g * 16, 16)] = w
                plsc.addupdate_scatter(rs_v, [svec >> 7, svec & 127], w)

        def finish(c, b):
            del c
            idx_v, sd_v, rows_v, wc_v, sem_i, sem_g, sem_s = bufs[b]

            pltpu.make_async_copy(h_r.at[sd_v.at[1]], rows_v, sem_g).wait()

            @plsc.parallel_loop(0, K // 16)
            def scale(g):
                wv = wc_v[pl.ds(g * 16, 16)]
                for l in range(16):
                    k = g * 16 + l
                    w = lax.gather(
                        wv, jnp.full((16, 1), l, jnp.int32),
                        lax.GatherDimensionNumbers(
                            offset_dims=(), collapsed_slice_dims=(0,),
                            start_index_map=(0,)),
                        (1,), mode=lax.GatherScatterMode.PROMISE_IN_BOUNDS)
                    for j in range(D // 16):
                        rows_v[k, pl.ds(j * 16, 16)] = (
                            rows_v[k, pl.ds(j * 16, 16)] * w)

            pltpu.async_copy(rows_v, acc_sh.at[sd_v.at[0]], sem_s, add=True)

        prepare(0, 0, steady=False)
        prepare(1, 1, steady=False)

        def pair(p, c):
            c0 = p * 2
            finish(c0, 0)
            prepare(c0 + 2, 0, steady=True)
            finish(c0 + 1, 1)
            prepare(c0 + 3, 1, steady=True)
            return c
        lax.fori_loop(0, NCHUNK // 2 - 1, pair, 0)

        finish(NCHUNK - 3, 0)
        prepare(NCHUNK - 1, 0, steady=True)
        finish(NCHUNK - 2, 1)
        finish(NCHUNK - 1, 0)

        pltpu.make_async_copy(h_r.at[pl.ds(0, K)], rows0_v, sem_s0).wait()
        pltpu.make_async_copy(h_r.at[pl.ds(0, K)], rows1_v, sem_s1).wait()

        for g in range(RSROWS // 16):
            rowids = lax.iota(jnp.int32, 16) + g * 16
            pltpu.sync_copy(rs_v.at[pl.ds(g * 16, 16)],
                            rs_sh.at[rowids], add=True)

        plsc.subcore_barrier()

        @pl.when(sid == 0)
        def _():
            pltpu.sync_copy(rs_sh, rsp_r.at[cid])

        @pl.when(sid < NS - 1)
        def _():
            base = pl.multiple_of(sid * RPT, 8)
            pltpu.sync_copy(acc_sh.at[pl.ds(base, RPT)],
                            parts_r.at[cid, pl.ds(base, RPT)])

        @pl.when(sid == NS - 1)
        def _():
            pltpu.sync_copy(acc_sh.at[pl.ds((NS - 1) * RPT, RPT_LAST)],
                            parts_r.at[cid, pl.ds((NS - 1) * RPT, RPT_LAST)])

    f = pl.kernel(
        body,
        out_type=[
            jax.ShapeDtypeStruct((NC, N, D), jnp.float32),
            jax.ShapeDtypeStruct((NC, RSROWS, D), jnp.float32),
        ],
        mesh=mesh,
        scratch_types=[
            pltpu.VMEM((N,), jnp.int32),
            pltpu.VMEM((K, D), jnp.float32),
            pltpu.VMEM((K, D), jnp.float32),
            pltpu.VMEM((RSROWS, D), jnp.float32),
            pltpu.VMEM((K,), jnp.float32),
            pltpu.VMEM((K,), jnp.float32),
            pltpu.VMEM((2 * K,), jnp.int32),
            pltpu.VMEM((2 * K,), jnp.int32),
            pltpu.VMEM((2, K), jnp.int32),
            pltpu.VMEM((2, K), jnp.int32),
            pltpu.VMEM_SHARED((N, D), jnp.float32),
            pltpu.VMEM_SHARED((RSROWS, D), jnp.float32),
            pltpu.SemaphoreType.DMA,
            pltpu.SemaphoreType.DMA,
            pltpu.SemaphoreType.DMA,
            pltpu.SemaphoreType.DMA,
            pltpu.SemaphoreType.DMA,
            pltpu.SemaphoreType.DMA,
        ],
        compiler_params=pltpu.CompilerParams(needs_layout_passes=False),
    )
    return f(h, eidx, f12, zrs)


def _stage3_body(p0_ref, p1_ref, r0_ref, r1_ref, out_ref):
    hp = p0_ref[...] + p1_ref[...]
    rs = r0_ref[...] + r1_ref[...] + 1e-9
    r = hp / rs
    out_ref[...] = jnp.where(r > 0, r, jnp.exp(jnp.minimum(r, 0.0)) - 1.0)


def _stage3(p0, p1, r0, r1):
    blk = 2000
    return pl.pallas_call(
        _stage3_body,
        grid=(N // blk,),
        in_specs=[
            pl.BlockSpec((blk, D), lambda i: (i, 0)),
            pl.BlockSpec((blk, D), lambda i: (i, 0)),
            pl.BlockSpec((blk, 1), lambda i: (i, 0)),
            pl.BlockSpec((blk, 1), lambda i: (i, 0)),
        ],
        out_specs=pl.BlockSpec((blk, D), lambda i: (i, 0)),
        out_shape=jax.ShapeDtypeStruct((N, D), jnp.float32),
    )(p0, p1, r0, r1)


def kernel(input, edge_index, W, a):
    eidx = jnp.swapaxes(
        edge_index.reshape(2, NW * NCHUNK, K), 0, 1).reshape(-1)
    h, f12 = _stage1(input, W, a)
    zrs = jnp.zeros((RPT, D), jnp.float32)
    parts, rsp = _stage2(h, eidx, f12.reshape(N), zrs)
    rs = rsp.reshape(NC, RSROWS * D)[:, :N].reshape(NC, N, 1)
    return _stage3(parts[0], parts[1], rs[0], rs[1])


# speedup vs baseline: 1.3594x; 1.0337x over previous
import jax
import jax.numpy as jnp
from jax import lax
from jax.experimental import pallas as pl
from jax.experimental.pallas import tpu as pltpu
from jax.experimental.pallas import tpu_sc as plsc

N = 10000
E = 320000
D = 128
ALPHA = 0.2

NC = 2
NS = 16
NW = NC * NS
EPW = E // NW
K = 80
NCHUNK = EPW // K
RPT = 640
RPT_LAST = N - (NS - 1) * RPT


def _stage1_body(x_ref, w_ref, a_ref, h_ref, f12_ref):
    h = jnp.dot(x_ref[...], w_ref[...], preferred_element_type=jnp.float32)
    h_ref[...] = h
    a1 = a_ref[0:1, :D]
    a2 = a_ref[0:1, D:]
    f1 = jnp.sum(h * a1, axis=1, keepdims=True)
    f2 = jnp.sum(h * a2, axis=1, keepdims=True)
    u1 = lax.bitcast_convert_type(
        f1.astype(jnp.bfloat16), jnp.uint16).astype(jnp.uint32)
    u2 = lax.bitcast_convert_type(
        f2.astype(jnp.bfloat16), jnp.uint16).astype(jnp.uint32)
    f12_ref[...] = lax.bitcast_convert_type((u2 << 16) | u1, jnp.int32)


def _stage1(x, W, a):
    blk = 2000
    return pl.pallas_call(
        _stage1_body,
        grid=(N // blk,),
        in_specs=[
            pl.BlockSpec((blk, D), lambda i: (i, 0)),
            pl.BlockSpec((D, D), lambda i: (0, 0)),
            pl.BlockSpec((1, 2 * D), lambda i: (0, 0)),
        ],
        out_specs=[
            pl.BlockSpec((blk, D), lambda i: (i, 0)),
            pl.BlockSpec((blk, 1), lambda i: (i, 0)),
        ],
        out_shape=[
            jax.ShapeDtypeStruct((N, D), jnp.float32),
            jax.ShapeDtypeStruct((N, 1), jnp.int32),
        ],
    )(x, W, a)


RSROWS = 80


def _stage2(h, eidx, f12, zrs):
    mesh = plsc.VectorSubcoreMesh(core_axis_name="c", subcore_axis_name="s")

    def body(h_r, eidx_r, f12_r, zrs_r, parts_r, rsp_r,
             f12_v, rows0_v, rows1_v, rs_v, wc0_v, wc1_v, idx0_v, idx1_v,
             sd0_v, sd1_v, acc_sh, rs_sh,
             sem_i0, sem_i1, sem_g0, sem_g1, sem_s0, sem_s1):
        cid = lax.axis_index("c")
        sid = lax.axis_index("s")
        wid = cid * NS + sid

        pltpu.sync_copy(f12_r, f12_v)

        @pl.when(sid < NS - 1)
        def _():
            base = pl.multiple_of(sid * RPT, 8)
            pltpu.sync_copy(zrs_r, acc_sh.at[pl.ds(base, RPT)])

        @pl.when(sid == NS - 1)
        def _():
            pltpu.sync_copy(zrs_r.at[pl.ds(0, RPT_LAST)],
                            acc_sh.at[pl.ds((NS - 1) * RPT, RPT_LAST)])

        @pl.when(sid == 0)
        def _():
            pltpu.sync_copy(zrs_r.at[pl.ds(0, RSROWS)], rs_sh)

        zero16 = jnp.zeros((16,), jnp.float32)

        def zbody(i, c):
            for j in range(D // 16):
                rs_v[i, pl.ds(j * 16, 16)] = zero16
            return c
        lax.fori_loop(0, RSROWS, zbody, 0)

        ebase0 = wid * EPW * 2

        def fire_fill(c, idx_v, sem_i):
            off = pl.multiple_of(ebase0 + c * 2 * K, 8)
            return pltpu.async_copy(
                eidx_r.at[pl.ds(off, 2 * K)], idx_v, sem_i)

        fire_fill(0, idx0_v, sem_i0)
        fire_fill(1, idx1_v, sem_i1)

        plsc.subcore_barrier()

        bufs = [
            (idx0_v, sd0_v, rows0_v, wc0_v, sem_i0, sem_g0, sem_s0),
            (idx1_v, sd1_v, rows1_v, wc1_v, sem_i1, sem_g1, sem_s1),
        ]

        def prepare(c, b, steady):
            idx_v, sd_v, rows_v, wc_v, sem_i, sem_g, sem_s = bufs[b]

            pltpu.make_async_copy(
                eidx_r.at[pl.ds(0, 2 * K)], idx_v, sem_i).wait()

            if steady:
                pltpu.make_async_copy(
                    h_r.at[pl.ds(0, K)], rows_v, sem_s).wait()

            for g in range(K // 16):
                sd_v[0, pl.ds(g * 16, 16)] = idx_v[pl.ds(g * 16, 16)]
                sd_v[1, pl.ds(g * 16, 16)] = idx_v[pl.ds(K + g * 16, 16)]

            @pl.when(c < NCHUNK - 2)
            def _():
                fire_fill(c + 2, idx_v, sem_i)

            pltpu.async_copy(h_r.at[sd_v.at[1]], rows_v, sem_g)

            for g in range(K // 16):
                svec = sd_v[0, pl.ds(g * 16, 16)]
                dvec = sd_v[1, pl.ds(g * 16, 16)]
                gs = plsc.load_gather(f12_v, [svec])
                gd = plsc.load_gather(f12_v, [dvec])
                f1 = plsc.bitcast(gs << 16, jnp.float32)
                f2 = plsc.bitcast(gd & jnp.int32(-65536), jnp.float32)
                z = f1 + f2
                w = jnp.exp(-jnp.where(z > 0, z, ALPHA * z))
                wc_v[pl.ds(g * 16, 16)] = w
                plsc.addupdate_scatter(rs_v, [svec >> 7, svec & 127], w)

        def finish(c, b):
            del c
            idx_v, sd_v, rows_v, wc_v, sem_i, sem_g, sem_s = bufs[b]

            pltpu.make_async_copy(h_r.at[sd_v.at[1]], rows_v, sem_g).wait()

            @plsc.parallel_loop(0, K // 16)
            def scale(g):
                wv = wc_v[pl.ds(g * 16, 16)]
                for l in range(16):
                    k = g * 16 + l
                    w = lax.gather(
                        wv, jnp.full((16, 1), l, jnp.int32),
                        lax.GatherDimensionNumbers(
                            offset_dims=(), collapsed_slice_dims=(0,),
                            start_index_map=(0,)),
                        (1,), mode=lax.GatherScatterMode.PROMISE_IN_BOUNDS)
                    for j in range(D // 16):
                        rows_v[k, pl.ds(j * 16, 16)] = (
                            rows_v[k, pl.ds(j * 16, 16)] * w)

            pltpu.async_copy(rows_v, acc_sh.at[sd_v.at[0]], sem_s, add=True)

        prepare(0, 0, steady=False)
        prepare(1, 1, steady=False)

        def pair(p, c):
            c0 = p * 2
            finish(c0, 0)
            prepare(c0 + 2, 0, steady=True)
            finish(c0 + 1, 1)
            prepare(c0 + 3, 1, steady=True)
            return c
        lax.fori_loop(0, NCHUNK // 2 - 1, pair, 0)

        finish(NCHUNK - 3, 0)
        prepare(NCHUNK - 1, 0, steady=True)
        finish(NCHUNK - 2, 1)
        finish(NCHUNK - 1, 0)

        pltpu.make_async_copy(h_r.at[pl.ds(0, K)], rows0_v, sem_s0).wait()
        pltpu.make_async_copy(h_r.at[pl.ds(0, K)], rows1_v, sem_s1).wait()

        for g in range(RSROWS // 16):
            rowids = lax.iota(jnp.int32, 16) + g * 16
            pltpu.sync_copy(rs_v.at[pl.ds(g * 16, 16)],
                            rs_sh.at[rowids], add=True)

        plsc.subcore_barrier()

        @pl.when(sid == 0)
        def _():
            pltpu.sync_copy(rs_sh, rsp_r.at[cid])

        @pl.when(sid < NS - 1)
        def _():
            base = pl.multiple_of(sid * RPT, 8)
            pltpu.sync_copy(acc_sh.at[pl.ds(base, RPT)],
                            parts_r.at[cid, pl.ds(base, RPT)])

        @pl.when(sid == NS - 1)
        def _():
            pltpu.sync_copy(acc_sh.at[pl.ds((NS - 1) * RPT, RPT_LAST)],
                            parts_r.at[cid, pl.ds((NS - 1) * RPT, RPT_LAST)])

    f = pl.kernel(
        body,
        out_type=[
            jax.ShapeDtypeStruct((NC, N, D), jnp.float32),
            jax.ShapeDtypeStruct((NC, RSROWS, D), jnp.float32),
        ],
        mesh=mesh,
        scratch_types=[
            pltpu.VMEM((N,), jnp.int32),
            pltpu.VMEM((K, D), jnp.float32),
            pltpu.VMEM((K, D), jnp.float32),
            pltpu.VMEM((RSROWS, D), jnp.float32),
            pltpu.VMEM((K,), jnp.float32),
            pltpu.VMEM((K,), jnp.float32),
            pltpu.VMEM((2 * K,), jnp.int32),
            pltpu.VMEM((2 * K,), jnp.int32),
            pltpu.VMEM((2, K), jnp.int32),
            pltpu.VMEM((2, K), jnp.int32),
            pltpu.VMEM_SHARED((N, D), jnp.float32),
            pltpu.VMEM_SHARED((RSROWS, D), jnp.float32),
            pltpu.SemaphoreType.DMA,
            pltpu.SemaphoreType.DMA,
            pltpu.SemaphoreType.DMA,
            pltpu.SemaphoreType.DMA,
            pltpu.SemaphoreType.DMA,
            pltpu.SemaphoreType.DMA,
        ],
        compiler_params=pltpu.CompilerParams(needs_layout_passes=False),
    )
    return f(h, eidx, f12, zrs)


def _stage3_body(p0_ref, p1_ref, r0_ref, r1_ref, out_ref):
    hp = p0_ref[0] + p1_ref[0]
    rs = r0_ref[0] + r1_ref[0] + 1e-9
    r = hp / rs
    out_ref[...] = jnp.where(r > 0, r, jnp.exp(jnp.minimum(r, 0.0)) - 1.0)


def _stage3(parts, rs2):
    blk = 2000
    return pl.pallas_call(
        _stage3_body,
        grid=(N // blk,),
        in_specs=[
            pl.BlockSpec((1, blk, D), lambda i: (0, i, 0)),
            pl.BlockSpec((1, blk, D), lambda i: (1, i, 0)),
            pl.BlockSpec((1, blk, 1), lambda i: (0, i, 0)),
            pl.BlockSpec((1, blk, 1), lambda i: (1, i, 0)),
        ],
        out_specs=pl.BlockSpec((blk, D), lambda i: (i, 0)),
        out_shape=jax.ShapeDtypeStruct((N, D), jnp.float32),
    )(parts, parts, rs2, rs2)


def kernel(input, edge_index, W, a):
    eidx = jnp.swapaxes(
        edge_index.reshape(2, NW * NCHUNK, K), 0, 1).reshape(-1)
    h, f12 = _stage1(input, W, a)
    zrs = jnp.zeros((RPT, D), jnp.float32)
    parts, rsp = _stage2(h, eidx, f12.reshape(N), zrs)
    rs2 = rsp.reshape(NC, RSROWS * D)[:, :N].reshape(NC, N, 1)
    return _stage3(parts, rs2)


# speedup vs baseline: 1.3940x; 1.0255x over previous
import jax
import jax.numpy as jnp
from jax import lax
from jax.experimental import pallas as pl
from jax.experimental.pallas import tpu as pltpu
from jax.experimental.pallas import tpu_sc as plsc

N = 10000
E = 320000
D = 128
ALPHA = 0.2

NC = 2
NS = 16
NW = NC * NS
EPW = E // NW
K = 80
NCHUNK = EPW // K
RPT = 640
RPT_LAST = N - (NS - 1) * RPT


def _stage1_body(x_ref, w_ref, a_ref, h_ref, f12_ref):
    h = jnp.dot(x_ref[...], w_ref[...], preferred_element_type=jnp.float32)
    h_ref[...] = h
    a1 = a_ref[0:1, :D]
    a2 = a_ref[0:1, D:]
    f1 = jnp.sum(h * a1, axis=1, keepdims=True)
    f2 = jnp.sum(h * a2, axis=1, keepdims=True)
    u1 = lax.bitcast_convert_type(
        f1.astype(jnp.bfloat16), jnp.uint16).astype(jnp.uint32)
    u2 = lax.bitcast_convert_type(
        f2.astype(jnp.bfloat16), jnp.uint16).astype(jnp.uint32)
    f12_ref[...] = lax.bitcast_convert_type((u2 << 16) | u1, jnp.int32)


def _stage1(x, W, a):
    blk = 2000
    return pl.pallas_call(
        _stage1_body,
        grid=(N // blk,),
        in_specs=[
            pl.BlockSpec((blk, D), lambda i: (i, 0)),
            pl.BlockSpec((D, D), lambda i: (0, 0)),
            pl.BlockSpec((1, 2 * D), lambda i: (0, 0)),
        ],
        out_specs=[
            pl.BlockSpec((blk, D), lambda i: (i, 0)),
            pl.BlockSpec((blk, 1), lambda i: (i, 0)),
        ],
        out_shape=[
            jax.ShapeDtypeStruct((N, D), jnp.float32),
            jax.ShapeDtypeStruct((N, 1), jnp.int32),
        ],
    )(x, W, a)


RSROWS = 80


def _stage2(h, eidx, f12):
    mesh = plsc.VectorSubcoreMesh(core_axis_name="c", subcore_axis_name="s")

    def body(h_r, eidx_r, f12_r, parts_r, rsp_r,
             f12_v, rows0_v, rows1_v, rs_v, wc0_v, wc1_v, idx0_v, idx1_v,
             sd0_v, sd1_v, acc_sh, rs_sh,
             sem_i0, sem_i1, sem_g0, sem_g1, sem_s0, sem_s1):
        cid = lax.axis_index("c")
        sid = lax.axis_index("s")
        wid = cid * NS + sid

        pltpu.sync_copy(f12_r, f12_v)

        zero16 = jnp.zeros((16,), jnp.float32)

        def zbody(i, c):
            for j in range(D // 16):
                rs_v[i, pl.ds(j * 16, 16)] = zero16
            return c
        lax.fori_loop(0, RSROWS, zbody, 0)

        @pl.when(sid < NS - 1)
        def _():
            for r in range(RPT // RSROWS):
                base = pl.multiple_of(sid * RPT + r * RSROWS, 8)
                pltpu.sync_copy(rs_v, acc_sh.at[pl.ds(base, RSROWS)])

        @pl.when(sid == NS - 1)
        def _():
            for r in range(RPT_LAST // RSROWS):
                base = (NS - 1) * RPT + r * RSROWS
                pltpu.sync_copy(rs_v, acc_sh.at[pl.ds(base, RSROWS)])

        @pl.when(sid == 0)
        def _():
            pltpu.sync_copy(rs_v, rs_sh)

        ebase0 = wid * EPW * 2

        def fire_fill(c, idx_v, sem_i):
            off = pl.multiple_of(ebase0 + c * 2 * K, 8)
            return pltpu.async_copy(
                eidx_r.at[pl.ds(off, 2 * K)], idx_v, sem_i)

        fire_fill(0, idx0_v, sem_i0)
        fire_fill(1, idx1_v, sem_i1)

        plsc.subcore_barrier()

        bufs = [
            (idx0_v, sd0_v, rows0_v, wc0_v, sem_i0, sem_g0, sem_s0),
            (idx1_v, sd1_v, rows1_v, wc1_v, sem_i1, sem_g1, sem_s1),
        ]

        def prepare(c, b, steady):
            idx_v, sd_v, rows_v, wc_v, sem_i, sem_g, sem_s = bufs[b]

            pltpu.make_async_copy(
                eidx_r.at[pl.ds(0, 2 * K)], idx_v, sem_i).wait()

            if steady:
                pltpu.make_async_copy(
                    h_r.at[pl.ds(0, K)], rows_v, sem_s).wait()

            for g in range(K // 16):
                sd_v[0, pl.ds(g * 16, 16)] = idx_v[pl.ds(g * 16, 16)]
                sd_v[1, pl.ds(g * 16, 16)] = idx_v[pl.ds(K + g * 16, 16)]

            @pl.when(c < NCHUNK - 2)
            def _():
                fire_fill(c + 2, idx_v, sem_i)

            pltpu.async_copy(h_r.at[sd_v.at[1]], rows_v, sem_g)

            for g in range(K // 16):
                svec = sd_v[0, pl.ds(g * 16, 16)]
                dvec = sd_v[1, pl.ds(g * 16, 16)]
                gs = plsc.load_gather(f12_v, [svec])
                gd = plsc.load_gather(f12_v, [dvec])
                f1 = plsc.bitcast(gs << 16, jnp.float32)
                f2 = plsc.bitcast(gd & jnp.int32(-65536), jnp.float32)
                z = f1 + f2
                w = jnp.exp(-jnp.where(z > 0, z, ALPHA * z))
                wc_v[pl.ds(g * 16, 16)] = w
                plsc.addupdate_scatter(rs_v, [svec >> 7, svec & 127], w)

        def finish(c, b):
            del c
            idx_v, sd_v, rows_v, wc_v, sem_i, sem_g, sem_s = bufs[b]

            pltpu.make_async_copy(h_r.at[sd_v.at[1]], rows_v, sem_g).wait()

            @plsc.parallel_loop(0, K // 16)
            def scale(g):
                wv = wc_v[pl.ds(g * 16, 16)]
                for l in range(16):
                    k = g * 16 + l
                    w = lax.gather(
                        wv, jnp.full((16, 1), l, jnp.int32),
                        lax.GatherDimensionNumbers(
                            offset_dims=(), collapsed_slice_dims=(0,),
                            start_index_map=(0,)),
                        (1,), mode=lax.GatherScatterMode.PROMISE_IN_BOUNDS)
                    for j in range(D // 16):
                        rows_v[k, pl.ds(j * 16, 16)] = (
                            rows_v[k, pl.ds(j * 16, 16)] * w)

            pltpu.async_copy(rows_v, acc_sh.at[sd_v.at[0]], sem_s, add=True)

        prepare(0, 0, steady=False)
        prepare(1, 1, steady=False)

        def pair(p, c):
            c0 = p * 2
            finish(c0, 0)
            prepare(c0 + 2, 0, steady=True)
            finish(c0 + 1, 1)
            prepare(c0 + 3, 1, steady=True)
            return c
        lax.fori_loop(0, NCHUNK // 2 - 1, pair, 0)

        finish(NCHUNK - 3, 0)
        prepare(NCHUNK - 1, 0, steady=True)
        finish(NCHUNK - 2, 1)
        finish(NCHUNK - 1, 0)

        pltpu.make_async_copy(h_r.at[pl.ds(0, K)], rows0_v, sem_s0).wait()
        pltpu.make_async_copy(h_r.at[pl.ds(0, K)], rows1_v, sem_s1).wait()

        for g in range(RSROWS // 16):
            rowids = lax.iota(jnp.int32, 16) + g * 16
            pltpu.sync_copy(rs_v.at[pl.ds(g * 16, 16)],
                            rs_sh.at[rowids], add=True)

        plsc.subcore_barrier()

        @pl.when(sid == 0)
        def _():
            pltpu.sync_copy(rs_sh, rsp_r.at[cid])

        @pl.when(sid < NS - 1)
        def _():
            base = pl.multiple_of(sid * RPT, 8)
            pltpu.sync_copy(acc_sh.at[pl.ds(base, RPT)],
                            parts_r.at[cid, pl.ds(base, RPT)])

        @pl.when(sid == NS - 1)
        def _():
            pltpu.sync_copy(acc_sh.at[pl.ds((NS - 1) * RPT, RPT_LAST)],
                            parts_r.at[cid, pl.ds((NS - 1) * RPT, RPT_LAST)])

    f = pl.kernel(
        body,
        out_type=[
            jax.ShapeDtypeStruct((NC, N, D), jnp.float32),
            jax.ShapeDtypeStruct((NC, RSROWS, D), jnp.float32),
        ],
        mesh=mesh,
        scratch_types=[
            pltpu.VMEM((N,), jnp.int32),
            pltpu.VMEM((K, D), jnp.float32),
            pltpu.VMEM((K, D), jnp.float32),
            pltpu.VMEM((RSROWS, D), jnp.float32),
            pltpu.VMEM((K,), jnp.float32),
            pltpu.VMEM((K,), jnp.float32),
            pltpu.VMEM((2 * K,), jnp.int32),
            pltpu.VMEM((2 * K,), jnp.int32),
            pltpu.VMEM((2, K), jnp.int32),
            pltpu.VMEM((2, K), jnp.int32),
            pltpu.VMEM_SHARED((N, D), jnp.float32),
            pltpu.VMEM_SHARED((RSROWS, D), jnp.float32),
            pltpu.SemaphoreType.DMA,
            pltpu.SemaphoreType.DMA,
            pltpu.SemaphoreType.DMA,
            pltpu.SemaphoreType.DMA,
            pltpu.SemaphoreType.DMA,
            pltpu.SemaphoreType.DMA,
        ],
        compiler_params=pltpu.CompilerParams(needs_layout_passes=False),
    )
    return f(h, eidx, f12)


def _stage3_body(p0_ref, p1_ref, r0_ref, r1_ref, out_ref):
    hp = p0_ref[0] + p1_ref[0]
    rs = r0_ref[0] + r1_ref[0] + 1e-9
    r = hp / rs
    out_ref[...] = jnp.where(r > 0, r, jnp.exp(jnp.minimum(r, 0.0)) - 1.0)


def _stage3(parts, rs2):
    blk = 2000
    return pl.pallas_call(
        _stage3_body,
        grid=(N // blk,),
        in_specs=[
            pl.BlockSpec((1, blk, D), lambda i: (0, i, 0)),
            pl.BlockSpec((1, blk, D), lambda i: (1, i, 0)),
            pl.BlockSpec((1, blk, 1), lambda i: (0, i, 0)),
            pl.BlockSpec((1, blk, 1), lambda i: (1, i, 0)),
        ],
        out_specs=pl.BlockSpec((blk, D), lambda i: (i, 0)),
        out_shape=jax.ShapeDtypeStruct((N, D), jnp.float32),
    )(parts, parts, rs2, rs2)


def kernel(input, edge_index, W, a):
    eidx = jnp.swapaxes(
        edge_index.reshape(2, NW * NCHUNK, K), 0, 1).reshape(-1)
    h, f12 = _stage1(input, W, a)
    parts, rsp = _stage2(h, eidx, f12.reshape(N))
    rs2 = rsp.reshape(NC, RSROWS * D)[:, :N].reshape(NC, N, 1)
    return _stage3(parts, rs2)


# speedup vs baseline: 1.4741x; 1.0575x over previous
import jax
import jax.numpy as jnp
from jax import lax
from jax.experimental import pallas as pl
from jax.experimental.pallas import tpu as pltpu
from jax.experimental.pallas import tpu_sc as plsc

N = 10000
E = 320000
D = 128
ALPHA = 0.2

NC = 2
NS = 16
NW = NC * NS
EPW = E // NW
K = 80
NCHUNK = EPW // K
RPT = 640
RPT_LAST = N - (NS - 1) * RPT


def _stage1_body(x_ref, w_ref, a_ref, h_ref, f12_ref):
    h = jnp.dot(x_ref[...], w_ref[...], preferred_element_type=jnp.float32)
    h_ref[...] = h
    a1 = a_ref[0:1, :D]
    a2 = a_ref[0:1, D:]
    f1 = jnp.sum(h * a1, axis=1, keepdims=True)
    f2 = jnp.sum(h * a2, axis=1, keepdims=True)
    u1 = lax.bitcast_convert_type(
        f1.astype(jnp.bfloat16), jnp.uint16).astype(jnp.uint32)
    u2 = lax.bitcast_convert_type(
        f2.astype(jnp.bfloat16), jnp.uint16).astype(jnp.uint32)
    f12_ref[...] = lax.bitcast_convert_type((u2 << 16) | u1, jnp.int32)


def _stage1(x, W, a):
    blk = 2000
    return pl.pallas_call(
        _stage1_body,
        grid=(N // blk,),
        in_specs=[
            pl.BlockSpec((blk, D), lambda i: (i, 0)),
            pl.BlockSpec((D, D), lambda i: (0, 0)),
            pl.BlockSpec((1, 2 * D), lambda i: (0, 0)),
        ],
        out_specs=[
            pl.BlockSpec((blk, D), lambda i: (i, 0)),
            pl.BlockSpec((blk, 1), lambda i: (i, 0)),
        ],
        out_shape=[
            jax.ShapeDtypeStruct((N, D), jnp.float32),
            jax.ShapeDtypeStruct((N, 1), jnp.int32),
        ],
    )(x, W, a)


RSROWS = 80


def _stage2(h, srcf, dstf, f12):
    mesh = plsc.VectorSubcoreMesh(core_axis_name="c", subcore_axis_name="s")

    def body(h_r, src_r, dst_r, f12_r, parts_r, rsp_r,
             f12_v, rows0_v, rows1_v, rs_v, wc0_v, wc1_v, idx0_v, idx1_v,
             sd0_v, sd1_v, acc_sh, rs_sh,
             sem_i0, sem_i1, sem_g0, sem_g1, sem_s0, sem_s1):
        cid = lax.axis_index("c")
        sid = lax.axis_index("s")
        wid = cid * NS + sid

        pltpu.sync_copy(f12_r, f12_v)

        zero16 = jnp.zeros((16,), jnp.float32)

        def zbody(i, c):
            for j in range(D // 16):
                rs_v[i, pl.ds(j * 16, 16)] = zero16
            return c
        lax.fori_loop(0, RSROWS, zbody, 0)

        @pl.when(sid < NS - 1)
        def _():
            for r in range(RPT // RSROWS):
                base = pl.multiple_of(sid * RPT + r * RSROWS, 8)
                pltpu.sync_copy(rs_v, acc_sh.at[pl.ds(base, RSROWS)])

        @pl.when(sid == NS - 1)
        def _():
            for r in range(RPT_LAST // RSROWS):
                base = (NS - 1) * RPT + r * RSROWS
                pltpu.sync_copy(rs_v, acc_sh.at[pl.ds(base, RSROWS)])

        @pl.when(sid == 0)
        def _():
            pltpu.sync_copy(rs_v, rs_sh)

        ebase0 = wid * EPW

        def fire_fill(c, idx_v, sem_i):
            off = pl.multiple_of(ebase0 + c * K, 8)
            pltpu.async_copy(src_r.at[pl.ds(off, K)],
                             idx_v.at[pl.ds(0, K)], sem_i)
            pltpu.async_copy(dst_r.at[pl.ds(off, K)],
                             idx_v.at[pl.ds(K, K)], sem_i)

        fire_fill(0, idx0_v, sem_i0)
        fire_fill(1, idx1_v, sem_i1)

        plsc.subcore_barrier()

        bufs = [
            (idx0_v, sd0_v, rows0_v, wc0_v, sem_i0, sem_g0, sem_s0),
            (idx1_v, sd1_v, rows1_v, wc1_v, sem_i1, sem_g1, sem_s1),
        ]

        def prepare(c, b, steady):
            idx_v, sd_v, rows_v, wc_v, sem_i, sem_g, sem_s = bufs[b]

            pltpu.make_async_copy(
                src_r.at[pl.ds(0, 2 * K)], idx_v, sem_i).wait()

            if steady:
                pltpu.make_async_copy(
                    h_r.at[pl.ds(0, K)], rows_v, sem_s).wait()

            for g in range(K // 16):
                sd_v[0, pl.ds(g * 16, 16)] = idx_v[pl.ds(g * 16, 16)]
                sd_v[1, pl.ds(g * 16, 16)] = idx_v[pl.ds(K + g * 16, 16)]

            @pl.when(c < NCHUNK - 2)
            def _():
                fire_fill(c + 2, idx_v, sem_i)

            pltpu.async_copy(h_r.at[sd_v.at[1]], rows_v, sem_g)

            for g in range(K // 16):
                svec = sd_v[0, pl.ds(g * 16, 16)]
                dvec = sd_v[1, pl.ds(g * 16, 16)]
                gs = plsc.load_gather(f12_v, [svec])
                gd = plsc.load_gather(f12_v, [dvec])
                f1 = plsc.bitcast(gs << 16, jnp.float32)
                f2 = plsc.bitcast(gd & jnp.int32(-65536), jnp.float32)
                z = f1 + f2
                w = jnp.exp(-jnp.where(z > 0, z, ALPHA * z))
                wc_v[pl.ds(g * 16, 16)] = w
                plsc.addupdate_scatter(rs_v, [svec >> 7, svec & 127], w)

        def finish(c, b):
            del c
            idx_v, sd_v, rows_v, wc_v, sem_i, sem_g, sem_s = bufs[b]

            pltpu.make_async_copy(h_r.at[sd_v.at[1]], rows_v, sem_g).wait()

            @plsc.parallel_loop(0, K // 16)
            def scale(g):
                wv = wc_v[pl.ds(g * 16, 16)]
                for l in range(16):
                    k = g * 16 + l
                    w = lax.gather(
                        wv, jnp.full((16, 1), l, jnp.int32),
                        lax.GatherDimensionNumbers(
                            offset_dims=(), collapsed_slice_dims=(0,),
                            start_index_map=(0,)),
                        (1,), mode=lax.GatherScatterMode.PROMISE_IN_BOUNDS)
                    for j in range(D // 16):
                        rows_v[k, pl.ds(j * 16, 16)] = (
                            rows_v[k, pl.ds(j * 16, 16)] * w)

            pltpu.async_copy(rows_v, acc_sh.at[sd_v.at[0]], sem_s, add=True)

        prepare(0, 0, steady=False)
        prepare(1, 1, steady=False)

        def pair(p, c):
            c0 = p * 2
            finish(c0, 0)
            prepare(c0 + 2, 0, steady=True)
            finish(c0 + 1, 1)
            prepare(c0 + 3, 1, steady=True)
            return c
        lax.fori_loop(0, NCHUNK // 2 - 1, pair, 0)

        finish(NCHUNK - 3, 0)
        prepare(NCHUNK - 1, 0, steady=True)
        finish(NCHUNK - 2, 1)
        finish(NCHUNK - 1, 0)

        pltpu.make_async_copy(h_r.at[pl.ds(0, K)], rows0_v, sem_s0).wait()
        pltpu.make_async_copy(h_r.at[pl.ds(0, K)], rows1_v, sem_s1).wait()

        for g in range(RSROWS // 16):
            rowids = lax.iota(jnp.int32, 16) + g * 16
            pltpu.sync_copy(rs_v.at[pl.ds(g * 16, 16)],
                            rs_sh.at[rowids], add=True)

        plsc.subcore_barrier()

        @pl.when(sid == 0)
        def _():
            pltpu.sync_copy(rs_sh, rsp_r.at[cid])

        @pl.when(sid < NS - 1)
        def _():
            base = pl.multiple_of(sid * RPT, 8)
            pltpu.sync_copy(acc_sh.at[pl.ds(base, RPT)],
                            parts_r.at[cid, pl.ds(base, RPT)])

        @pl.when(sid == NS - 1)
        def _():
            pltpu.sync_copy(acc_sh.at[pl.ds((NS - 1) * RPT, RPT_LAST)],
                            parts_r.at[cid, pl.ds((NS - 1) * RPT, RPT_LAST)])

    f = pl.kernel(
        body,
        out_type=[
            jax.ShapeDtypeStruct((NC, N, D), jnp.float32),
            jax.ShapeDtypeStruct((NC, RSROWS, D), jnp.float32),
        ],
        mesh=mesh,
        scratch_types=[
            pltpu.VMEM((N,), jnp.int32),
            pltpu.VMEM((K, D), jnp.float32),
            pltpu.VMEM((K, D), jnp.float32),
            pltpu.VMEM((RSROWS, D), jnp.float32),
            pltpu.VMEM((K,), jnp.float32),
            pltpu.VMEM((K,), jnp.float32),
            pltpu.VMEM((2 * K,), jnp.int32),
            pltpu.VMEM((2 * K,), jnp.int32),
            pltpu.VMEM((2, K), jnp.int32),
            pltpu.VMEM((2, K), jnp.int32),
            pltpu.VMEM_SHARED((N, D), jnp.float32),
            pltpu.VMEM_SHARED((RSROWS, D), jnp.float32),
            pltpu.SemaphoreType.DMA,
            pltpu.SemaphoreType.DMA,
            pltpu.SemaphoreType.DMA,
            pltpu.SemaphoreType.DMA,
            pltpu.SemaphoreType.DMA,
            pltpu.SemaphoreType.DMA,
        ],
        compiler_params=pltpu.CompilerParams(needs_layout_passes=False),
    )
    return f(h, srcf, dstf, f12)


def _stage3_body(p0_ref, p1_ref, r0_ref, r1_ref, out_ref):
    hp = p0_ref[0] + p1_ref[0]
    rs = r0_ref[0] + r1_ref[0] + 1e-9
    r = hp / rs
    out_ref[...] = jnp.where(r > 0, r, jnp.exp(jnp.minimum(r, 0.0)) - 1.0)


def _stage3(parts, rs2):
    blk = 2000
    return pl.pallas_call(
        _stage3_body,
        grid=(N // blk,),
        in_specs=[
            pl.BlockSpec((1, blk, D), lambda i: (0, i, 0)),
            pl.BlockSpec((1, blk, D), lambda i: (1, i, 0)),
            pl.BlockSpec((1, blk, 1), lambda i: (0, i, 0)),
            pl.BlockSpec((1, blk, 1), lambda i: (1, i, 0)),
        ],
        out_specs=pl.BlockSpec((blk, D), lambda i: (i, 0)),
        out_shape=jax.ShapeDtypeStruct((N, D), jnp.float32),
    )(parts, parts, rs2, rs2)


def kernel(input, edge_index, W, a):
    h, f12 = _stage1(input, W, a)
    parts, rsp = _stage2(h, edge_index[0], edge_index[1], f12.reshape(N))
    rs2 = rsp.reshape(NC, RSROWS * D)[:, :N].reshape(NC, N, 1)
    return _stage3(parts, rs2)
